# Initial kernel scaffold; baseline (speedup 1.0000x reference)
import jax
import jax.numpy as jnp
from jax.experimental import pallas as pl


def kernel(x, homo_edge_index, W0, b0):
    raise NotImplementedError("write your pallas kernel here")


# speedup vs baseline: 15.9682x; 15.9682x over previous
import functools

import jax
import jax.numpy as jnp
from jax import lax
from jax.experimental import pallas as pl
from jax.experimental.pallas import tpu as pltpu
from jax.experimental.pallas import tpu_sc as plsc

_NUM_SC = 2
_NUM_TILES = 16
_CH = 80


def _mesh():
    return plsc.VectorSubcoreMesh(core_axis_name="c", subcore_axis_name="s")


def _tc_matmul(x, w):
    def body(x_ref, w_ref, o_ref):
        o_ref[...] = lax.dot(
            x_ref[...], w_ref[...],
            precision=lax.Precision.HIGHEST,
            preferred_element_type=jnp.float32,
        )

    return pl.pallas_call(
        body,
        out_shape=jax.ShapeDtypeStruct((x.shape[0], w.shape[1]), jnp.float32),
    )(x, w)


def _sc_degree(row, col, n):
    e = row.shape[0]
    epw = e // (_NUM_SC * _NUM_TILES)
    nch = epw // _CH
    pad = 10240 if n == 10000 else ((n + _NUM_TILES * 128) //
                                    (_NUM_TILES * 128)) * _NUM_TILES * 128
    zpt = pad // _NUM_TILES // 128

    @functools.partial(
        pl.kernel,
        out_type=[jax.ShapeDtypeStruct((pad,), jnp.float32),
                  jax.ShapeDtypeStruct((pad,), jnp.float32)],
        mesh=_mesh(),
        scratch_types=[
            pltpu.VMEM_SHARED((pad,), jnp.float32),
            pltpu.VMEM((1, _CH), jnp.int32),
            pltpu.VMEM((1, _CH), jnp.int32),
            pltpu.VMEM((1, _CH), jnp.int32),
            pltpu.VMEM((_CH,), jnp.float32),
            pltpu.VMEM((128,), jnp.float32),
        ],
    )
    def deg_kernel(row_hbm, col_hbm, out0_hbm, out1_hbm, dacc, rbuf,
                   cbuf, ibuf, ones, stage):
        cid = lax.axis_index("c")
        sid = lax.axis_index("s")

        @pl.loop(0, 128 // 16)
        def _(i):
            stage[pl.ds(i * 16, 16)] = jnp.zeros((16,), jnp.float32)

        @pl.loop(0, _CH // 16)
        def _(i):
            ones[pl.ds(i * 16, 16)] = jnp.full((16,), 1.0, jnp.float32)

        @pl.loop(0, zpt)
        def _(k):
            pltpu.sync_copy(stage,
                            dacc.at[pl.ds((sid * zpt + k) * 128, 128)])

        plsc.subcore_barrier()
        base = (cid * _NUM_TILES + sid) * epw

        @pl.loop(0, nch)
        def _(j):
            off = base + j * _CH
            pltpu.sync_copy(row_hbm.at[pl.ds(off, _CH)], rbuf.at[0])
            pltpu.sync_copy(col_hbm.at[pl.ds(off, _CH)], cbuf.at[0])

            @pl.loop(0, _CH // 16)
            def _(i):
                r = rbuf[0, pl.ds(i * 16, 16)]
                c = cbuf[0, pl.ds(i * 16, 16)]
                ibuf[0, pl.ds(i * 16, 16)] = jnp.where(r == c, n, r)

            pltpu.sync_copy(ones, dacc.at[ibuf.at[0]], add=True)

        plsc.subcore_barrier()

        @pl.loop(0, zpt)
        def _(k):
            off = (sid * zpt + k) * 128
            pltpu.sync_copy(dacc.at[pl.ds(off, 128)], stage)

            @pl.when(cid == 0)
            def _():
                pltpu.sync_copy(stage, out0_hbm.at[pl.ds(off, 128)])

            @pl.when(cid == 1)
            def _():
                pltpu.sync_copy(stage, out1_hbm.at[pl.ds(off, 128)])

    return deg_kernel(row, col)


def _tc_scale(d0c, d1c, xw):
    def body(d0_ref, d1_ref, xw_ref, y_ref):
        deg = d0_ref[...] + d1_ref[...] + 1.0
        dinv = lax.rsqrt(deg)
        y_ref[...] = xw_ref[...] * dinv

    return pl.pallas_call(
        body,
        out_shape=jax.ShapeDtypeStruct(xw.shape, jnp.float32),
    )(d0c, d1c, xw)


def _sc_scatter(row, col, y):
    e = row.shape[0]
    n, d = y.shape
    epw = e // (_NUM_SC * _NUM_TILES)
    nch = epw // _CH
    pad = 10240 if n == 10000 else ((n + _NUM_TILES * _CH) //
                                    (_NUM_TILES * _CH)) * _NUM_TILES * _CH
    zpt = pad // _NUM_TILES // _CH
    wch = -(-(n // _CH) // _NUM_TILES)

    @functools.partial(
        pl.kernel,
        out_type=[jax.ShapeDtypeStruct((n, d), jnp.float32),
                  jax.ShapeDtypeStruct((n, d), jnp.float32)],
        mesh=_mesh(),
        scratch_types=[
            pltpu.VMEM_SHARED((pad, d), jnp.float32),
            pltpu.VMEM((1, _CH), jnp.int32),
            pltpu.VMEM((1, _CH), jnp.int32),
            pltpu.VMEM((1, _CH), jnp.int32),
            pltpu.VMEM((_CH, d), jnp.float32),
        ],
    )
    def main_kernel(row_hbm, col_hbm, y_hbm, out0_hbm, out1_hbm, acc,
                    rbuf, cbuf, ibuf, rows):
        cid = lax.axis_index("c")
        sid = lax.axis_index("s")

        @pl.loop(0, _CH)
        def _(r):
            @pl.loop(0, d // 16)
            def _(i):
                rows[r, pl.ds(i * 16, 16)] = jnp.zeros((16,), jnp.float32)

        @pl.loop(0, zpt)
        def _(k):
            pltpu.sync_copy(
                rows, acc.at[pl.ds((sid * zpt + k) * _CH, _CH)])

        plsc.subcore_barrier()
        base = (cid * _NUM_TILES + sid) * epw

        @pl.loop(0, nch)
        def _(j):
            off = base + j * _CH
            pltpu.sync_copy(row_hbm.at[pl.ds(off, _CH)], rbuf.at[0])
            pltpu.sync_copy(col_hbm.at[pl.ds(off, _CH)], cbuf.at[0])

            @pl.loop(0, _CH // 16)
            def _(i):
                r = rbuf[0, pl.ds(i * 16, 16)]
                c = cbuf[0, pl.ds(i * 16, 16)]
                ibuf[0, pl.ds(i * 16, 16)] = jnp.where(r == c, n, c)

            pltpu.sync_copy(y_hbm.at[rbuf.at[0]], rows)
            pltpu.sync_copy(rows, acc.at[ibuf.at[0]], add=True)

        plsc.subcore_barrier()

        @pl.loop(0, wch)
        def _(k):
            ch = sid + k * _NUM_TILES

            @pl.when(ch * _CH < n)
            def _():
                pltpu.sync_copy(acc.at[pl.ds(ch * _CH, _CH)], rows)

                @pl.when(cid == 0)
                def _():
                    pltpu.sync_copy(rows, out0_hbm.at[pl.ds(ch * _CH, _CH)])

                @pl.when(cid == 1)
                def _():
                    pltpu.sync_copy(rows, out1_hbm.at[pl.ds(ch * _CH, _CH)])

    return main_kernel(row, col, y)


def _tc_finish(d0c, d1c, y, acc0, acc1, b):
    def body(d0_ref, d1_ref, y_ref, a0_ref, a1_ref, b_ref, o_ref):
        deg = d0_ref[...] + d1_ref[...] + 1.0
        dinv = lax.rsqrt(deg)
        s = a0_ref[...] + a1_ref[...] + y_ref[...]
        o_ref[...] = jnp.maximum(s * dinv + b_ref[...], 0.0)

    return pl.pallas_call(
        body,
        out_shape=jax.ShapeDtypeStruct(y.shape, jnp.float32),
    )(d0c, d1c, y, acc0, acc1, b)


def kernel(x, homo_edge_index, W0, b0):
    n = x.shape[0]
    row = homo_edge_index[0]
    col = homo_edge_index[1]
    xw = _tc_matmul(x, W0)
    deg0, deg1 = _sc_degree(row, col, n)
    d0c = deg0[:n].reshape(n, 1)
    d1c = deg1[:n].reshape(n, 1)
    y = _tc_scale(d0c, d1c, xw)
    acc0, acc1 = _sc_scatter(row, col, y)
    return _tc_finish(d0c, d1c, y, acc0, acc1, b0.reshape(1, -1))


# speedup vs baseline: 45.0918x; 2.8238x over previous
import functools

import jax
import jax.numpy as jnp
from jax import lax
from jax.experimental import pallas as pl
from jax.experimental.pallas import tpu as pltpu
from jax.experimental.pallas import tpu_sc as plsc

_NUM_SC = 2
_NUM_TILES = 16
_NW = _NUM_SC * _NUM_TILES
_CH = 80
_NBUF = 3
_IBUF = 6


def _mesh():
    return plsc.VectorSubcoreMesh(core_axis_name="c", subcore_axis_name="s")


def _pad_rows(n):
    return ((n + 1 + _NUM_TILES * _CH - 1) // (_NUM_TILES * _CH)) \
        * _NUM_TILES * _CH


def _tc_matmul(x, w, row3, col3, n):
    def body(x_ref, w_ref, r_ref, c_ref, o_ref, cp_ref):
        o_ref[...] = lax.dot(
            x_ref[...], w_ref[...],
            precision=lax.Precision.HIGHEST,
            preferred_element_type=jnp.float32,
        )
        r = r_ref[...]
        c = c_ref[...]
        cp_ref[...] = jnp.where(r == c, n, c)

    return pl.pallas_call(
        body,
        out_shape=[
            jax.ShapeDtypeStruct((x.shape[0], w.shape[1]), jnp.float32),
            jax.ShapeDtypeStruct(col3.shape, jnp.int32),
        ],
    )(x, w, row3, col3)


def _sc_degree(row3, col3, n):
    nch = row3.shape[1]
    pad = _pad_rows(n)
    zpt = pad // _NUM_TILES // 128

    @functools.partial(
        pl.kernel,
        out_type=[jax.ShapeDtypeStruct((pad,), jnp.float32),
                  jax.ShapeDtypeStruct((pad,), jnp.float32)],
        mesh=_mesh(),
        scratch_types=[
            pltpu.VMEM_SHARED((pad,), jnp.float32),
            pltpu.VMEM((nch, _CH), jnp.int32),
            pltpu.VMEM((nch, _CH), jnp.int32),
            pltpu.VMEM((nch, _CH), jnp.int32),
            pltpu.VMEM((_CH,), jnp.float32),
            pltpu.VMEM((128,), jnp.float32),
            pltpu.SemaphoreType.DMA,
        ],
    )
    def deg_kernel(row_hbm, col_hbm, out0_hbm, out1_hbm, dacc, rbuf,
                   cbuf, ibuf, ones, stage, sem):
        cid = lax.axis_index("c")
        sid = lax.axis_index("s")
        wid = cid * _NUM_TILES + sid

        @pl.loop(0, 128 // 16)
        def _(i):
            stage[pl.ds(i * 16, 16)] = jnp.zeros((16,), jnp.float32)

        @pl.loop(0, _CH // 16)
        def _(i):
            ones[pl.ds(i * 16, 16)] = jnp.full((16,), 1.0, jnp.float32)

        pltpu.async_copy(row_hbm.at[wid], rbuf, sem)
        pltpu.async_copy(col_hbm.at[wid], cbuf, sem)

        @pl.loop(0, zpt)
        def _(k):
            pltpu.sync_copy(stage,
                            dacc.at[pl.ds((sid * zpt + k) * 128, 128)])

        pltpu.make_async_copy(row_hbm.at[wid], rbuf, sem).wait()
        pltpu.make_async_copy(col_hbm.at[wid], cbuf, sem).wait()

        @pl.loop(0, nch)
        def _(j):
            @pl.loop(0, _CH // 16)
            def _(i):
                r = rbuf[j, pl.ds(i * 16, 16)]
                c = cbuf[j, pl.ds(i * 16, 16)]
                ibuf[j, pl.ds(i * 16, 16)] = jnp.where(r == c, n, r)

        plsc.subcore_barrier()

        @pl.loop(0, nch)
        def _(j):
            pltpu.async_copy(ones, dacc.at[ibuf.at[j]], sem, add=True)

        @pl.loop(0, nch)
        def _(j):
            pltpu.make_async_copy(ones, dacc.at[ibuf.at[j]], sem).wait()

        plsc.subcore_barrier()

        @pl.loop(0, zpt)
        def _(k):
            off = (sid * zpt + k) * 128
            pltpu.sync_copy(dacc.at[pl.ds(off, 128)], stage)

            @pl.when(cid == 0)
            def _():
                pltpu.sync_copy(stage, out0_hbm.at[pl.ds(off, 128)])

            @pl.when(cid == 1)
            def _():
                pltpu.sync_copy(stage, out1_hbm.at[pl.ds(off, 128)])

    return deg_kernel(row3, col3)


def _tc_scale(d0c, d1c, xw):
    def body(d0_ref, d1_ref, xw_ref, y_ref):
        deg = d0_ref[...] + d1_ref[...] + 1.0
        dinv = lax.rsqrt(deg)
        y_ref[...] = xw_ref[...] * dinv

    return pl.pallas_call(
        body,
        out_shape=jax.ShapeDtypeStruct(xw.shape, jnp.float32),
    )(d0c, d1c, xw)


def _sc_scatter(row3, colp3, y):
    nch = row3.shape[1]
    n, d = y.shape
    pad = _pad_rows(n)
    zpt = pad // _NUM_TILES // _CH
    wch = -(-(n // _CH) // _NUM_TILES)

    @functools.partial(
        pl.kernel,
        out_type=[jax.ShapeDtypeStruct((n, d), jnp.float32),
                  jax.ShapeDtypeStruct((n, d), jnp.float32)],
        mesh=_mesh(),
        scratch_types=[
            pltpu.VMEM_SHARED((pad, d), jnp.float32),
            pltpu.VMEM((_IBUF, _CH), jnp.int32),
            pltpu.VMEM((_IBUF, _CH), jnp.int32),
            pltpu.VMEM((_NBUF, _CH, d), jnp.float32),
            pltpu.SemaphoreType.DMA,
            pltpu.SemaphoreType.DMA,
            pltpu.SemaphoreType.DMA,
        ],
    )
    def main_kernel(row_hbm, colp_hbm, y_hbm, out0_hbm, out1_hbm, acc,
                    rbuf, cbuf, rows, isem, gsem, ssem):
        cid = lax.axis_index("c")
        sid = lax.axis_index("s")
        wid = cid * _NUM_TILES + sid

        def i_issue(j, bi):
            pltpu.async_copy(row_hbm.at[wid, j], rbuf.at[bi], isem)
            pltpu.async_copy(colp_hbm.at[wid, j], cbuf.at[bi], isem)

        def i_wait(j, bi):
            pltpu.make_async_copy(row_hbm.at[wid, j], rbuf.at[bi],
                                  isem).wait()
            pltpu.make_async_copy(colp_hbm.at[wid, j], cbuf.at[bi],
                                  isem).wait()

        def g_issue(bi, b):
            pltpu.async_copy(y_hbm.at[rbuf.at[bi]], rows.at[b], gsem)

        def g_wait(bi, b):
            pltpu.make_async_copy(y_hbm.at[rbuf.at[bi]], rows.at[b],
                                  gsem).wait()

        def s_issue(bi, b):
            pltpu.async_copy(rows.at[b], acc.at[cbuf.at[bi]], ssem, add=True)

        def s_wait(bi, b):
            pltpu.make_async_copy(rows.at[b], acc.at[cbuf.at[bi]],
                                  ssem).wait()

        for j in range(4):
            i_issue(j, j)

        @pl.loop(0, _CH)
        def _(r):
            @pl.loop(0, d // 16)
            def _(i):
                rows[0, r, pl.ds(i * 16, 16)] = jnp.zeros((16,), jnp.float32)

        @pl.loop(0, zpt)
        def _(k):
            pltpu.sync_copy(
                rows.at[0], acc.at[pl.ds((sid * zpt + k) * _CH, _CH)])

        plsc.subcore_barrier()

        i_wait(0, 0)
        g_issue(0, 0)
        i_wait(1, 1)
        g_issue(1, 1)

        @pl.loop(0, nch)
        def _(j):
            b = lax.rem(j, _NBUF)
            bi = lax.rem(j, _IBUF)
            g_wait(bi, b)
            s_issue(bi, b)

            @pl.when(j >= 1)
            def _():
                s_wait(lax.rem(j - 1, _IBUF), lax.rem(j - 1, _NBUF))

            @pl.when(j + 2 < nch)
            def _():
                bi2 = lax.rem(j + 2, _IBUF)
                i_wait(j + 2, bi2)
                g_issue(bi2, lax.rem(j + 2, _NBUF))

            @pl.when(j + 4 < nch)
            def _():
                i_issue(j + 4, lax.rem(j + 4, _IBUF))

        s_wait(lax.rem(nch - 1, _IBUF), lax.rem(nch - 1, _NBUF))
        plsc.subcore_barrier()

        @pl.loop(0, wch)
        def _(k):
            ch = sid + k * _NUM_TILES

            @pl.when(ch * _CH < n)
            def _():
                pltpu.sync_copy(acc.at[pl.ds(ch * _CH, _CH)], rows.at[0])

                @pl.when(cid == 0)
                def _():
                    pltpu.sync_copy(rows.at[0],
                                    out0_hbm.at[pl.ds(ch * _CH, _CH)])

                @pl.when(cid == 1)
                def _():
                    pltpu.sync_copy(rows.at[0],
                                    out1_hbm.at[pl.ds(ch * _CH, _CH)])

    return main_kernel(row3, colp3, y)


def _tc_finish(d0c, d1c, y, acc0, acc1, b):
    def body(d0_ref, d1_ref, y_ref, a0_ref, a1_ref, b_ref, o_ref):
        deg = d0_ref[...] + d1_ref[...] + 1.0
        dinv = lax.rsqrt(deg)
        s = a0_ref[...] + a1_ref[...] + y_ref[...]
        o_ref[...] = jnp.maximum(s * dinv + b_ref[...], 0.0)

    return pl.pallas_call(
        body,
        out_shape=jax.ShapeDtypeStruct(y.shape, jnp.float32),
    )(d0c, d1c, y, acc0, acc1, b)


def kernel(x, homo_edge_index, W0, b0):
    n = x.shape[0]
    e = homo_edge_index.shape[1]
    nch = e // (_NW * _CH)
    row3 = homo_edge_index[0].reshape(_NW, nch, _CH)
    col3 = homo_edge_index[1].reshape(_NW, nch, _CH)
    xw, colp3 = _tc_matmul(x, W0, row3, col3, n)
    deg0, deg1 = _sc_degree(row3, col3, n)
    d0c = deg0[:n].reshape(n, 1)
    d1c = deg1[:n].reshape(n, 1)
    y = _tc_scale(d0c, d1c, xw)
    acc0, acc1 = _sc_scatter(row3, colp3, y)
    return _tc_finish(d0c, d1c, y, acc0, acc1, b0.reshape(1, -1))


# speedup vs baseline: 47.4998x; 1.0534x over previous
import functools

import jax
import jax.numpy as jnp
from jax import lax
from jax.experimental import pallas as pl
from jax.experimental.pallas import tpu as pltpu
from jax.experimental.pallas import tpu_sc as plsc

_NUM_SC = 2
_NUM_TILES = 16
_NW = _NUM_SC * _NUM_TILES
_CH = 80
_NBUF = 4
_IBUF = 6


def _mesh():
    return plsc.VectorSubcoreMesh(core_axis_name="c", subcore_axis_name="s")


def _pad_rows(n):
    return ((n + 1 + _NUM_TILES * _CH - 1) // (_NUM_TILES * _CH)) \
        * _NUM_TILES * _CH


def _tc_matmul(x, w, row3, col3, n):
    def body(x_ref, w_ref, r_ref, c_ref, o_ref, cp_ref):
        o_ref[...] = lax.dot(
            x_ref[...], w_ref[...],
            precision=lax.Precision.HIGHEST,
            preferred_element_type=jnp.float32,
        )
        r = r_ref[...]
        c = c_ref[...]
        cp_ref[...] = jnp.where(r == c, n, c)

    return pl.pallas_call(
        body,
        out_shape=[
            jax.ShapeDtypeStruct((x.shape[0], w.shape[1]), jnp.float32),
            jax.ShapeDtypeStruct(col3.shape, jnp.int32),
        ],
    )(x, w, row3, col3)


def _sc_degree(row3, col3, n):
    nch = row3.shape[1]
    pad = _pad_rows(n)
    zpt = pad // _NUM_TILES // 128

    @functools.partial(
        pl.kernel,
        out_type=[jax.ShapeDtypeStruct((pad,), jnp.float32),
                  jax.ShapeDtypeStruct((pad,), jnp.float32)],
        mesh=_mesh(),
        scratch_types=[
            pltpu.VMEM_SHARED((pad,), jnp.float32),
            pltpu.VMEM((nch, _CH), jnp.int32),
            pltpu.VMEM((nch, _CH), jnp.int32),
            pltpu.VMEM((nch, _CH), jnp.int32),
            pltpu.VMEM((_CH,), jnp.float32),
            pltpu.VMEM((128,), jnp.float32),
            pltpu.SemaphoreType.DMA,
        ],
    )
    def deg_kernel(row_hbm, col_hbm, out0_hbm, out1_hbm, dacc, rbuf,
                   cbuf, ibuf, ones, stage, sem):
        cid = lax.axis_index("c")
        sid = lax.axis_index("s")
        wid = cid * _NUM_TILES + sid

        @pl.loop(0, 128 // 16)
        def _(i):
            stage[pl.ds(i * 16, 16)] = jnp.zeros((16,), jnp.float32)

        @pl.loop(0, _CH // 16)
        def _(i):
            ones[pl.ds(i * 16, 16)] = jnp.full((16,), 1.0, jnp.float32)

        pltpu.async_copy(row_hbm.at[wid], rbuf, sem)
        pltpu.async_copy(col_hbm.at[wid], cbuf, sem)

        @pl.loop(0, zpt)
        def _(k):
            pltpu.sync_copy(stage,
                            dacc.at[pl.ds((sid * zpt + k) * 128, 128)])

        pltpu.make_async_copy(row_hbm.at[wid], rbuf, sem).wait()
        pltpu.make_async_copy(col_hbm.at[wid], cbuf, sem).wait()

        @pl.loop(0, nch)
        def _(j):
            @pl.loop(0, _CH // 16)
            def _(i):
                r = rbuf[j, pl.ds(i * 16, 16)]
                c = cbuf[j, pl.ds(i * 16, 16)]
                ibuf[j, pl.ds(i * 16, 16)] = jnp.where(r == c, n, r)

        plsc.subcore_barrier()

        @pl.loop(0, nch)
        def _(j):
            pltpu.async_copy(ones, dacc.at[ibuf.at[j]], sem, add=True)

        @pl.loop(0, nch)
        def _(j):
            pltpu.make_async_copy(ones, dacc.at[ibuf.at[j]], sem).wait()

        plsc.subcore_barrier()

        @pl.loop(0, zpt)
        def _(k):
            off = (sid * zpt + k) * 128
            pltpu.sync_copy(dacc.at[pl.ds(off, 128)], stage)

            @pl.when(cid == 0)
            def _():
                pltpu.sync_copy(stage, out0_hbm.at[pl.ds(off, 128)])

            @pl.when(cid == 1)
            def _():
                pltpu.sync_copy(stage, out1_hbm.at[pl.ds(off, 128)])

    return deg_kernel(row3, col3)


def _tc_scale(d0c, d1c, xw):
    def body(d0_ref, d1_ref, xw_ref, y_ref):
        deg = d0_ref[...] + d1_ref[...] + 1.0
        dinv = lax.rsqrt(deg)
        y_ref[...] = xw_ref[...] * dinv

    return pl.pallas_call(
        body,
        out_shape=jax.ShapeDtypeStruct(xw.shape, jnp.float32),
    )(d0c, d1c, xw)


def _sc_scatter(row3, colp3, y):
    nch = row3.shape[1]
    n, d = y.shape
    pad = _pad_rows(n)
    zpt = pad // _NUM_TILES // _CH
    wch = -(-(n // _CH) // _NUM_TILES)

    @functools.partial(
        pl.kernel,
        out_type=[jax.ShapeDtypeStruct((n, d), jnp.float32),
                  jax.ShapeDtypeStruct((n, d), jnp.float32)],
        mesh=_mesh(),
        scratch_types=[
            pltpu.VMEM_SHARED((pad, d), jnp.float32),
            pltpu.VMEM((_IBUF, _CH), jnp.int32),
            pltpu.VMEM((_IBUF, _CH), jnp.int32),
            pltpu.VMEM((_NBUF, _CH, d), jnp.float32),
            pltpu.SemaphoreType.DMA,
            pltpu.SemaphoreType.DMA,
            pltpu.SemaphoreType.DMA,
        ],
    )
    def main_kernel(row_hbm, colp_hbm, y_hbm, out0_hbm, out1_hbm, acc,
                    rbuf, cbuf, rows, isem, gsem, ssem):
        cid = lax.axis_index("c")
        sid = lax.axis_index("s")
        wid = cid * _NUM_TILES + sid

        def i_issue(j, bi):
            pltpu.async_copy(row_hbm.at[wid, j], rbuf.at[bi], isem)
            pltpu.async_copy(colp_hbm.at[wid, j], cbuf.at[bi], isem)

        def i_wait(j, bi):
            pltpu.make_async_copy(row_hbm.at[wid, j], rbuf.at[bi],
                                  isem).wait()
            pltpu.make_async_copy(colp_hbm.at[wid, j], cbuf.at[bi],
                                  isem).wait()

        def g_issue(bi, b):
            pltpu.async_copy(y_hbm.at[rbuf.at[bi]], rows.at[b], gsem)

        def g_wait(bi, b):
            pltpu.make_async_copy(y_hbm.at[rbuf.at[bi]], rows.at[b],
                                  gsem).wait()

        def s_issue(bi, b):
            pltpu.async_copy(rows.at[b], acc.at[cbuf.at[bi]], ssem, add=True)

        def s_wait(bi, b):
            pltpu.make_async_copy(rows.at[b], acc.at[cbuf.at[bi]],
                                  ssem).wait()

        for j in range(5):
            i_issue(j, j)

        @pl.loop(0, _CH)
        def _(r):
            @pl.loop(0, d // 16)
            def _(i):
                rows[0, r, pl.ds(i * 16, 16)] = jnp.zeros((16,), jnp.float32)

        @pl.loop(0, zpt)
        def _(k):
            pltpu.async_copy(
                rows.at[0], acc.at[pl.ds((sid * zpt + k) * _CH, _CH)], ssem)

        @pl.loop(0, zpt)
        def _(k):
            pltpu.make_async_copy(
                rows.at[0], acc.at[pl.ds((sid * zpt + k) * _CH, _CH)],
                ssem).wait()

        plsc.subcore_barrier()

        for j in range(3):
            i_wait(j, j)
            g_issue(j, j)

        @pl.loop(0, nch)
        def _(j):
            b = lax.rem(j, _NBUF)
            bi = lax.rem(j, _IBUF)
            g_wait(bi, b)
            s_issue(bi, b)

            @pl.when(j >= 1)
            def _():
                s_wait(lax.rem(j - 1, _IBUF), lax.rem(j - 1, _NBUF))

            @pl.when(j + 3 < nch)
            def _():
                bi3 = lax.rem(j + 3, _IBUF)
                i_wait(j + 3, bi3)
                g_issue(bi3, lax.rem(j + 3, _NBUF))

            @pl.when(j + 5 < nch)
            def _():
                i_issue(j + 5, lax.rem(j + 5, _IBUF))

        s_wait(lax.rem(nch - 1, _IBUF), lax.rem(nch - 1, _NBUF))
        plsc.subcore_barrier()

        @pl.loop(0, wch)
        def _(k):
            ch = sid + k * _NUM_TILES

            @pl.when(ch * _CH < n)
            def _():
                @pl.when(cid == 0)
                def _():
                    pltpu.async_copy(acc.at[pl.ds(ch * _CH, _CH)],
                                     out0_hbm.at[pl.ds(ch * _CH, _CH)], ssem)

                @pl.when(cid == 1)
                def _():
                    pltpu.async_copy(acc.at[pl.ds(ch * _CH, _CH)],
                                     out1_hbm.at[pl.ds(ch * _CH, _CH)], ssem)

        @pl.loop(0, wch)
        def _(k):
            ch = sid + k * _NUM_TILES

            @pl.when(ch * _CH < n)
            def _():
                @pl.when(cid == 0)
                def _():
                    pltpu.make_async_copy(
                        acc.at[pl.ds(ch * _CH, _CH)],
                        out0_hbm.at[pl.ds(ch * _CH, _CH)], ssem).wait()

                @pl.when(cid == 1)
                def _():
                    pltpu.make_async_copy(
                        acc.at[pl.ds(ch * _CH, _CH)],
                        out1_hbm.at[pl.ds(ch * _CH, _CH)], ssem).wait()

    return main_kernel(row3, colp3, y)


def _tc_finish(d0c, d1c, y, acc0, acc1, b):
    def body(d0_ref, d1_ref, y_ref, a0_ref, a1_ref, b_ref, o_ref):
        deg = d0_ref[...] + d1_ref[...] + 1.0
        dinv = lax.rsqrt(deg)
        s = a0_ref[...] + a1_ref[...] + y_ref[...]
        o_ref[...] = jnp.maximum(s * dinv + b_ref[...], 0.0)

    return pl.pallas_call(
        body,
        out_shape=jax.ShapeDtypeStruct(y.shape, jnp.float32),
    )(d0c, d1c, y, acc0, acc1, b)


def kernel(x, homo_edge_index, W0, b0):
    n = x.shape[0]
    e = homo_edge_index.shape[1]
    nch = e // (_NW * _CH)
    row3 = homo_edge_index[0].reshape(_NW, nch, _CH)
    col3 = homo_edge_index[1].reshape(_NW, nch, _CH)
    xw, colp3 = _tc_matmul(x, W0, row3, col3, n)
    deg0, deg1 = _sc_degree(row3, col3, n)
    d0c = deg0[:n].reshape(n, 1)
    d1c = deg1[:n].reshape(n, 1)
    y = _tc_scale(d0c, d1c, xw)
    acc0, acc1 = _sc_scatter(row3, colp3, y)
    return _tc_finish(d0c, d1c, y, acc0, acc1, b0.reshape(1, -1))


# speedup vs baseline: 48.9812x; 1.0312x over previous
import functools

import jax
import jax.numpy as jnp
from jax import lax
from jax.experimental import pallas as pl
from jax.experimental.pallas import tpu as pltpu
from jax.experimental.pallas import tpu_sc as plsc

_NUM_SC = 2
_NUM_TILES = 16
_NW = _NUM_SC * _NUM_TILES
_CH = 80
_NBUF = 4
_IBUF = 6


def _mesh():
    return plsc.VectorSubcoreMesh(core_axis_name="c", subcore_axis_name="s")


def _pad_rows(n):
    return ((n + 1 + _NUM_TILES * _CH - 1) // (_NUM_TILES * _CH)) \
        * _NUM_TILES * _CH


def _tc_matmul(x, w):
    def body(x_ref, w_ref, o_ref):
        o_ref[...] = lax.dot(
            x_ref[...], w_ref[...],
            precision=lax.Precision.HIGHEST,
            preferred_element_type=jnp.float32,
        )

    return pl.pallas_call(
        body,
        out_shape=jax.ShapeDtypeStruct((x.shape[0], w.shape[1]), jnp.float32),
    )(x, w)


def _sc_degree(row, col, n):
    e = row.shape[0]
    epw = e // _NW
    nch = epw // _CH
    pad = _pad_rows(n)
    zpt = pad // _NUM_TILES // 128

    @functools.partial(
        pl.kernel,
        out_type=[jax.ShapeDtypeStruct((pad,), jnp.float32),
                  jax.ShapeDtypeStruct((pad,), jnp.float32),
                  jax.ShapeDtypeStruct((e,), jnp.int32)],
        mesh=_mesh(),
        scratch_types=[
            pltpu.VMEM_SHARED((pad,), jnp.float32),
            pltpu.VMEM((nch, _CH), jnp.int32),
            pltpu.VMEM((nch, _CH), jnp.int32),
            pltpu.VMEM((nch, _CH), jnp.int32),
            pltpu.VMEM((nch, _CH), jnp.int32),
            pltpu.VMEM((_CH,), jnp.float32),
            pltpu.VMEM((128,), jnp.float32),
            pltpu.SemaphoreType.DMA,
            pltpu.SemaphoreType.DMA,
        ],
    )
    def deg_kernel(row_hbm, col_hbm, out0_hbm, out1_hbm, colp_hbm, dacc,
                   rbuf, cbuf, pbuf, ibuf, ones, stage, sem, psem):
        cid = lax.axis_index("c")
        sid = lax.axis_index("s")
        wid = cid * _NUM_TILES + sid
        base = wid * epw

        @pl.loop(0, nch)
        def _(j):
            pltpu.async_copy(row_hbm.at[pl.ds(base + j * _CH, _CH)],
                             rbuf.at[j], sem)
            pltpu.async_copy(col_hbm.at[pl.ds(base + j * _CH, _CH)],
                             cbuf.at[j], sem)

        @pl.loop(0, 128 // 16)
        def _(i):
            stage[pl.ds(i * 16, 16)] = jnp.zeros((16,), jnp.float32)

        @pl.loop(0, _CH // 16)
        def _(i):
            ones[pl.ds(i * 16, 16)] = jnp.full((16,), 1.0, jnp.float32)

        @pl.loop(0, zpt)
        def _(k):
            pltpu.sync_copy(stage,
                            dacc.at[pl.ds((sid * zpt + k) * 128, 128)])

        @pl.loop(0, nch)
        def _(j):
            pltpu.make_async_copy(row_hbm.at[pl.ds(base + j * _CH, _CH)],
                                  rbuf.at[j], sem).wait()
            pltpu.make_async_copy(col_hbm.at[pl.ds(base + j * _CH, _CH)],
                                  cbuf.at[j], sem).wait()

        @pl.loop(0, nch)
        def _(j):
            @pl.loop(0, _CH // 16)
            def _(i):
                sl = pl.ds(i * 16, 16)
                r = rbuf[j, sl]
                c = cbuf[j, sl]
                ibuf[j, sl] = jnp.where(r == c, n, r)
                pbuf[j, sl] = jnp.where(r == c, n, c)

        @pl.loop(0, nch)
        def _(j):
            pltpu.async_copy(pbuf.at[j],
                             colp_hbm.at[pl.ds(base + j * _CH, _CH)], psem)

        plsc.subcore_barrier()

        @pl.loop(0, nch)
        def _(j):
            pltpu.async_copy(ones, dacc.at[ibuf.at[j]], sem, add=True)

        @pl.loop(0, nch)
        def _(j):
            pltpu.make_async_copy(ones, dacc.at[ibuf.at[j]], sem).wait()

        @pl.loop(0, nch)
        def _(j):
            pltpu.make_async_copy(pbuf.at[j],
                                  colp_hbm.at[pl.ds(base + j * _CH, _CH)],
                                  psem).wait()

        plsc.subcore_barrier()

        @pl.loop(0, zpt)
        def _(k):
            off = (sid * zpt + k) * 128
            pltpu.sync_copy(dacc.at[pl.ds(off, 128)], stage)

            @pl.when(cid == 0)
            def _():
                pltpu.sync_copy(stage, out0_hbm.at[pl.ds(off, 128)])

            @pl.when(cid == 1)
            def _():
                pltpu.sync_copy(stage, out1_hbm.at[pl.ds(off, 128)])

    return deg_kernel(row, col)


def _tc_scale(d0c, d1c, xw):
    def body(d0_ref, d1_ref, xw_ref, y_ref):
        deg = d0_ref[...] + d1_ref[...] + 1.0
        dinv = lax.rsqrt(deg)
        y_ref[...] = xw_ref[...] * dinv

    return pl.pallas_call(
        body,
        out_shape=jax.ShapeDtypeStruct(xw.shape, jnp.float32),
    )(d0c, d1c, xw)


def _sc_scatter(row, colp, y):
    e = row.shape[0]
    n, d = y.shape
    epw = e // _NW
    nch = epw // _CH
    pad = _pad_rows(n)
    zpt = pad // _NUM_TILES // _CH
    wch = -(-(n // _CH) // _NUM_TILES)

    @functools.partial(
        pl.kernel,
        out_type=[jax.ShapeDtypeStruct((n, d), jnp.float32),
                  jax.ShapeDtypeStruct((n, d), jnp.float32)],
        mesh=_mesh(),
        scratch_types=[
            pltpu.VMEM_SHARED((pad, d), jnp.float32),
            pltpu.VMEM((_IBUF, _CH), jnp.int32),
            pltpu.VMEM((_IBUF, _CH), jnp.int32),
            pltpu.VMEM((_NBUF, _CH, d), jnp.float32),
            pltpu.SemaphoreType.DMA,
            pltpu.SemaphoreType.DMA,
            pltpu.SemaphoreType.DMA,
        ],
    )
    def main_kernel(row_hbm, colp_hbm, y_hbm, out0_hbm, out1_hbm, acc,
                    rbuf, cbuf, rows, isem, gsem, ssem):
        cid = lax.axis_index("c")
        sid = lax.axis_index("s")
        wid = cid * _NUM_TILES + sid
        base = wid * epw

        def i_issue(j, bi):
            pltpu.async_copy(row_hbm.at[pl.ds(base + j * _CH, _CH)],
                             rbuf.at[bi], isem)
            pltpu.async_copy(colp_hbm.at[pl.ds(base + j * _CH, _CH)],
                             cbuf.at[bi], isem)

        def i_wait(j, bi):
            pltpu.make_async_copy(row_hbm.at[pl.ds(base + j * _CH, _CH)],
                                  rbuf.at[bi], isem).wait()
            pltpu.make_async_copy(colp_hbm.at[pl.ds(base + j * _CH, _CH)],
                                  cbuf.at[bi], isem).wait()

        def g_issue(bi, b):
            pltpu.async_copy(y_hbm.at[rbuf.at[bi]], rows.at[b], gsem)

        def g_wait(bi, b):
            pltpu.make_async_copy(y_hbm.at[rbuf.at[bi]], rows.at[b],
                                  gsem).wait()

        def s_issue(bi, b):
            pltpu.async_copy(rows.at[b], acc.at[cbuf.at[bi]], ssem, add=True)

        def s_wait(bi, b):
            pltpu.make_async_copy(rows.at[b], acc.at[cbuf.at[bi]],
                                  ssem).wait()

        for j in range(5):
            i_issue(j, j)

        @pl.loop(0, _CH)
        def _(r):
            @pl.loop(0, d // 16)
            def _(i):
                rows[0, r, pl.ds(i * 16, 16)] = jnp.zeros((16,), jnp.float32)

        @pl.loop(0, zpt)
        def _(k):
            pltpu.async_copy(
                rows.at[0], acc.at[pl.ds((sid * zpt + k) * _CH, _CH)], ssem)

        @pl.loop(0, zpt)
        def _(k):
            pltpu.make_async_copy(
                rows.at[0], acc.at[pl.ds((sid * zpt + k) * _CH, _CH)],
                ssem).wait()

        plsc.subcore_barrier()

        for j in range(3):
            i_wait(j, j)
            g_issue(j, j)

        @pl.loop(0, nch)
        def _(j):
            b = lax.rem(j, _NBUF)
            bi = lax.rem(j, _IBUF)
            g_wait(bi, b)
            s_issue(bi, b)

            @pl.when(j >= 1)
            def _():
                s_wait(lax.rem(j - 1, _IBUF), lax.rem(j - 1, _NBUF))

            @pl.when(j + 3 < nch)
            def _():
                bi3 = lax.rem(j + 3, _IBUF)
                i_wait(j + 3, bi3)
                g_issue(bi3, lax.rem(j + 3, _NBUF))

            @pl.when(j + 5 < nch)
            def _():
                i_issue(j + 5, lax.rem(j + 5, _IBUF))

        s_wait(lax.rem(nch - 1, _IBUF), lax.rem(nch - 1, _NBUF))
        plsc.subcore_barrier()

        @pl.loop(0, wch)
        def _(k):
            ch = sid + k * _NUM_TILES

            @pl.when(ch * _CH < n)
            def _():
                @pl.when(cid == 0)
                def _():
                    pltpu.async_copy(acc.at[pl.ds(ch * _CH, _CH)],
                                     out0_hbm.at[pl.ds(ch * _CH, _CH)], ssem)

                @pl.when(cid == 1)
                def _():
                    pltpu.async_copy(acc.at[pl.ds(ch * _CH, _CH)],
                                     out1_hbm.at[pl.ds(ch * _CH, _CH)], ssem)

        @pl.loop(0, wch)
        def _(k):
            ch = sid + k * _NUM_TILES

            @pl.when(ch * _CH < n)
            def _():
                @pl.when(cid == 0)
                def _():
                    pltpu.make_async_copy(
                        acc.at[pl.ds(ch * _CH, _CH)],
                        out0_hbm.at[pl.ds(ch * _CH, _CH)], ssem).wait()

                @pl.when(cid == 1)
                def _():
                    pltpu.make_async_copy(
                        acc.at[pl.ds(ch * _CH, _CH)],
                        out1_hbm.at[pl.ds(ch * _CH, _CH)], ssem).wait()

    return main_kernel(row, colp, y)


def _tc_finish(d0c, d1c, y, acc0, acc1, b):
    def body(d0_ref, d1_ref, y_ref, a0_ref, a1_ref, b_ref, o_ref):
        deg = d0_ref[...] + d1_ref[...] + 1.0
        dinv = lax.rsqrt(deg)
        s = a0_ref[...] + a1_ref[...] + y_ref[...]
        o_ref[...] = jnp.maximum(s * dinv + b_ref[...], 0.0)

    return pl.pallas_call(
        body,
        out_shape=jax.ShapeDtypeStruct(y.shape, jnp.float32),
    )(d0c, d1c, y, acc0, acc1, b)


def kernel(x, homo_edge_index, W0, b0):
    n = x.shape[0]
    row = homo_edge_index[0]
    col = homo_edge_index[1]
    xw = _tc_matmul(x, W0)
    deg0, deg1, colp = _sc_degree(row, col, n)
    d0c = deg0[:n].reshape(n, 1)
    d1c = deg1[:n].reshape(n, 1)
    y = _tc_scale(d0c, d1c, xw)
    acc0, acc1 = _sc_scatter(row, colp, y)
    return _tc_finish(d0c, d1c, y, acc0, acc1, b0.reshape(1, -1))


# speedup vs baseline: 57.6692x; 1.1774x over previous
import functools

import jax
import jax.numpy as jnp
from jax import lax
from jax.experimental import pallas as pl
from jax.experimental.pallas import tpu as pltpu
from jax.experimental.pallas import tpu_sc as plsc

_NUM_SC = 2
_NUM_TILES = 16
_NW = _NUM_SC * _NUM_TILES
_CH = 80
_NBUF = 4
_IBUF = 6


def _mesh():
    return plsc.VectorSubcoreMesh(core_axis_name="c", subcore_axis_name="s")


def _pad_rows(n):
    return ((n + 1 + _NUM_TILES * _CH - 1) // (_NUM_TILES * _CH)) \
        * _NUM_TILES * _CH


def _tc_matmul(x, w):
    def body(x_ref, w_ref, o_ref):
        o_ref[...] = lax.dot(
            x_ref[...], w_ref[...],
            precision=lax.Precision.HIGHEST,
            preferred_element_type=jnp.float32,
        )

    return pl.pallas_call(
        body,
        out_shape=jax.ShapeDtypeStruct((x.shape[0], w.shape[1]), jnp.float32),
    )(x, w)


def _sc_degree(ei, n):
    e = ei.shape[0] // 2
    epw = e // _NW
    nch = epw // _CH
    pad = _pad_rows(n)
    zpt = pad // _NUM_TILES // 128

    @functools.partial(
        pl.kernel,
        out_type=[jax.ShapeDtypeStruct((pad,), jnp.float32),
                  jax.ShapeDtypeStruct((pad,), jnp.float32),
                  jax.ShapeDtypeStruct((e,), jnp.int32)],
        mesh=_mesh(),
        scratch_types=[
            pltpu.VMEM_SHARED((pad,), jnp.float32),
            pltpu.VMEM((nch, _CH), jnp.int32),
            pltpu.VMEM((nch, _CH), jnp.int32),
            pltpu.VMEM((nch, _CH), jnp.int32),
            pltpu.VMEM((nch, _CH), jnp.int32),
            pltpu.VMEM((_CH,), jnp.float32),
            pltpu.VMEM((128,), jnp.float32),
            pltpu.SemaphoreType.DMA,
            pltpu.SemaphoreType.DMA,
        ],
    )
    def deg_kernel(ei_hbm, out0_hbm, out1_hbm, colp_hbm, dacc,
                   rbuf, cbuf, pbuf, ibuf, ones, stage, sem, psem):
        cid = lax.axis_index("c")
        sid = lax.axis_index("s")
        wid = cid * _NUM_TILES + sid
        base = wid * epw

        @pl.loop(0, nch)
        def _(j):
            pltpu.async_copy(ei_hbm.at[pl.ds(base + j * _CH, _CH)],
                             rbuf.at[j], sem)
            pltpu.async_copy(ei_hbm.at[pl.ds(e + base + j * _CH, _CH)],
                             cbuf.at[j], sem)

        @pl.loop(0, 128 // 16)
        def _(i):
            stage[pl.ds(i * 16, 16)] = jnp.zeros((16,), jnp.float32)

        @pl.loop(0, _CH // 16)
        def _(i):
            ones[pl.ds(i * 16, 16)] = jnp.full((16,), 1.0, jnp.float32)

        @pl.loop(0, zpt)
        def _(k):
            pltpu.sync_copy(stage,
                            dacc.at[pl.ds((sid * zpt + k) * 128, 128)])

        @pl.loop(0, nch)
        def _(j):
            pltpu.make_async_copy(ei_hbm.at[pl.ds(base + j * _CH, _CH)],
                                  rbuf.at[j], sem).wait()
            pltpu.make_async_copy(ei_hbm.at[pl.ds(e + base + j * _CH, _CH)],
                                  cbuf.at[j], sem).wait()

        @pl.loop(0, nch)
        def _(j):
            @pl.loop(0, _CH // 16)
            def _(i):
                sl = pl.ds(i * 16, 16)
                r = rbuf[j, sl]
                c = cbuf[j, sl]
                ibuf[j, sl] = jnp.where(r == c, n, r)
                pbuf[j, sl] = jnp.where(r == c, n, c)

        @pl.loop(0, nch)
        def _(j):
            pltpu.async_copy(pbuf.at[j],
                             colp_hbm.at[pl.ds(base + j * _CH, _CH)], psem)

        plsc.subcore_barrier()

        @pl.loop(0, nch)
        def _(j):
            pltpu.async_copy(ones, dacc.at[ibuf.at[j]], sem, add=True)

        @pl.loop(0, nch)
        def _(j):
            pltpu.make_async_copy(ones, dacc.at[ibuf.at[j]], sem).wait()

        @pl.loop(0, nch)
        def _(j):
            pltpu.make_async_copy(pbuf.at[j],
                                  colp_hbm.at[pl.ds(base + j * _CH, _CH)],
                                  psem).wait()

        plsc.subcore_barrier()

        @pl.loop(0, zpt)
        def _(k):
            off = (sid * zpt + k) * 128
            pltpu.sync_copy(dacc.at[pl.ds(off, 128)], stage)

            @pl.when(cid == 0)
            def _():
                pltpu.sync_copy(stage, out0_hbm.at[pl.ds(off, 128)])

            @pl.when(cid == 1)
            def _():
                pltpu.sync_copy(stage, out1_hbm.at[pl.ds(off, 128)])

    return deg_kernel(ei)


def _tc_scale(d0, d1, xw):
    n = xw.shape[0]

    def body(d0_ref, d1_ref, xw_ref, y_ref, dc_ref):
        deg = d0_ref[...] + d1_ref[...] + 1.0
        dinv = lax.rsqrt(deg)
        dcol = dinv[:n].reshape(n, 1)
        dc_ref[...] = dcol
        y_ref[...] = xw_ref[...] * dcol

    return pl.pallas_call(
        body,
        out_shape=[jax.ShapeDtypeStruct(xw.shape, jnp.float32),
                   jax.ShapeDtypeStruct((n, 1), jnp.float32)],
    )(d0, d1, xw)


def _sc_scatter(ei, colp, y):
    e = ei.shape[0] // 2
    n, d = y.shape
    epw = e // _NW
    nch = epw // _CH
    pad = _pad_rows(n)
    zpt = pad // _NUM_TILES // _CH
    wch = -(-(n // _CH) // _NUM_TILES)

    @functools.partial(
        pl.kernel,
        out_type=[jax.ShapeDtypeStruct((n, d), jnp.float32),
                  jax.ShapeDtypeStruct((n, d), jnp.float32)],
        mesh=_mesh(),
        scratch_types=[
            pltpu.VMEM_SHARED((pad, d), jnp.float32),
            pltpu.VMEM((_IBUF, _CH), jnp.int32),
            pltpu.VMEM((_IBUF, _CH), jnp.int32),
            pltpu.VMEM((_NBUF, _CH, d), jnp.float32),
            pltpu.SemaphoreType.DMA,
            pltpu.SemaphoreType.DMA,
            pltpu.SemaphoreType.DMA,
        ],
    )
    def main_kernel(ei_hbm, colp_hbm, y_hbm, out0_hbm, out1_hbm, acc,
                    rbuf, cbuf, rows, isem, gsem, ssem):
        cid = lax.axis_index("c")
        sid = lax.axis_index("s")
        wid = cid * _NUM_TILES + sid
        base = wid * epw

        def i_issue(j, bi):
            pltpu.async_copy(ei_hbm.at[pl.ds(base + j * _CH, _CH)],
                             rbuf.at[bi], isem)
            pltpu.async_copy(colp_hbm.at[pl.ds(base + j * _CH, _CH)],
                             cbuf.at[bi], isem)

        def i_wait(j, bi):
            pltpu.make_async_copy(ei_hbm.at[pl.ds(base + j * _CH, _CH)],
                                  rbuf.at[bi], isem).wait()
            pltpu.make_async_copy(colp_hbm.at[pl.ds(base + j * _CH, _CH)],
                                  cbuf.at[bi], isem).wait()

        def g_issue(bi, b):
            pltpu.async_copy(y_hbm.at[rbuf.at[bi]], rows.at[b], gsem)

        def g_wait(bi, b):
            pltpu.make_async_copy(y_hbm.at[rbuf.at[bi]], rows.at[b],
                                  gsem).wait()

        def s_issue(bi, b):
            pltpu.async_copy(rows.at[b], acc.at[cbuf.at[bi]], ssem, add=True)

        def s_wait(bi, b):
            pltpu.make_async_copy(rows.at[b], acc.at[cbuf.at[bi]],
                                  ssem).wait()

        for j in range(5):
            i_issue(j, j)

        @pl.loop(0, _CH)
        def _(r):
            @pl.loop(0, d // 16)
            def _(i):
                rows[0, r, pl.ds(i * 16, 16)] = jnp.zeros((16,), jnp.float32)

        @pl.loop(0, zpt)
        def _(k):
            pltpu.async_copy(
                rows.at[0], acc.at[pl.ds((sid * zpt + k) * _CH, _CH)], ssem)

        @pl.loop(0, zpt)
        def _(k):
            pltpu.make_async_copy(
                rows.at[0], acc.at[pl.ds((sid * zpt + k) * _CH, _CH)],
                ssem).wait()

        plsc.subcore_barrier()

        for j in range(3):
            i_wait(j, j)
            g_issue(j, j)

        @pl.loop(0, nch)
        def _(j):
            b = lax.rem(j, _NBUF)
            bi = lax.rem(j, _IBUF)
            g_wait(bi, b)
            s_issue(bi, b)

            @pl.when(j >= 1)
            def _():
                s_wait(lax.rem(j - 1, _IBUF), lax.rem(j - 1, _NBUF))

            @pl.when(j + 3 < nch)
            def _():
                bi3 = lax.rem(j + 3, _IBUF)
                i_wait(j + 3, bi3)
                g_issue(bi3, lax.rem(j + 3, _NBUF))

            @pl.when(j + 5 < nch)
            def _():
                i_issue(j + 5, lax.rem(j + 5, _IBUF))

        s_wait(lax.rem(nch - 1, _IBUF), lax.rem(nch - 1, _NBUF))
        plsc.subcore_barrier()

        @pl.loop(0, wch)
        def _(k):
            ch = sid + k * _NUM_TILES

            @pl.when(ch * _CH < n)
            def _():
                @pl.when(cid == 0)
                def _():
                    pltpu.async_copy(acc.at[pl.ds(ch * _CH, _CH)],
                                     out0_hbm.at[pl.ds(ch * _CH, _CH)], ssem)

                @pl.when(cid == 1)
                def _():
                    pltpu.async_copy(acc.at[pl.ds(ch * _CH, _CH)],
                                     out1_hbm.at[pl.ds(ch * _CH, _CH)], ssem)

        @pl.loop(0, wch)
        def _(k):
            ch = sid + k * _NUM_TILES

            @pl.when(ch * _CH < n)
            def _():
                @pl.when(cid == 0)
                def _():
                    pltpu.make_async_copy(
                        acc.at[pl.ds(ch * _CH, _CH)],
                        out0_hbm.at[pl.ds(ch * _CH, _CH)], ssem).wait()

                @pl.when(cid == 1)
                def _():
                    pltpu.make_async_copy(
                        acc.at[pl.ds(ch * _CH, _CH)],
                        out1_hbm.at[pl.ds(ch * _CH, _CH)], ssem).wait()

    return main_kernel(ei, colp, y)


def _tc_finish(dc, y, acc0, acc1, b):
    def body(dc_ref, y_ref, a0_ref, a1_ref, b_ref, o_ref):
        s = a0_ref[...] + a1_ref[...] + y_ref[...]
        o_ref[...] = jnp.maximum(s * dc_ref[...] + b_ref[...], 0.0)

    return pl.pallas_call(
        body,
        out_shape=jax.ShapeDtypeStruct(y.shape, jnp.float32),
    )(dc, y, acc0, acc1, b)


def kernel(x, homo_edge_index, W0, b0):
    n = x.shape[0]
    eflat = homo_edge_index.reshape(-1)
    xw = _tc_matmul(x, W0)
    deg0, deg1, colp = _sc_degree(eflat, n)
    y, dc = _tc_scale(deg0, deg1, xw)
    acc0, acc1 = _sc_scatter(eflat, colp, y)
    return _tc_finish(dc, y, acc0, acc1, b0.reshape(1, -1))
